# baseline (speedup 1.0000x reference)
import functools

import jax
import jax.numpy as jnp
from jax.experimental import pallas as pl
from jax.experimental.pallas import tpu as pltpu

B, N, D, OUT = 4, 100000, 128, 10
TILE_N = 5000
NT = N // TILE_N


def _fused_kernel(h_ref, w_in_ref, w_h1_ref, w_h2_ref, w_out_ref,
                  c1w_ref, c1b_ref, c2w_ref, c2b_ref, c3w_ref, c3b_ref,
                  out_ref, pooled_ref):
    b = pl.program_id(0)
    nt = pl.program_id(1)

    @pl.when((b == 0) & (nt == 0))
    def _init():
        pooled_ref[:, :] = jnp.zeros((8, D), jnp.float32)

    x = h_ref[0]
    h = jnp.maximum(jnp.dot(x, w_in_ref[:, :], preferred_element_type=jnp.float32), 0.0)
    h = jnp.maximum(jnp.dot(h, w_h1_ref[:, :], preferred_element_type=jnp.float32), 0.0)
    h = jnp.maximum(jnp.dot(h, w_h2_ref[:, :], preferred_element_type=jnp.float32), 0.0)
    h = jnp.maximum(jnp.dot(h, w_out_ref[:, :], preferred_element_type=jnp.float32), 0.0)
    partial = jnp.sum(h, axis=0, keepdims=True)

    rows = jax.lax.broadcasted_iota(jnp.int32, (8, D), 0)
    pooled_ref[:, :] = jnp.where(rows == b, pooled_ref[:, :] + partial,
                                 pooled_ref[:, :])

    @pl.when((b == B - 1) & (nt == NT - 1))
    def _classify():
        acc = pooled_ref[0:B, :]
        y = jnp.maximum(jnp.dot(acc, c1w_ref[:, :],
                                preferred_element_type=jnp.float32)
                        + c1b_ref[:, :], 0.0)
        y = jnp.maximum(jnp.dot(y, c2w_ref[:, :],
                                preferred_element_type=jnp.float32)
                        + c2b_ref[:, :], 0.0)
        y = (jnp.dot(y, c3w_ref[:, :], preferred_element_type=jnp.float32)
             + c3b_ref[:, :])
        out_ref[:, :] = y


@functools.partial(jax.jit, static_argnames=())
def kernel(h_0, W_in, W_h1, W_h2, W_out, C1_w, C1_b, C2_w, C2_b, C3_w, C3_b):
    const = lambda shape: pl.BlockSpec(shape, lambda b, n: (0,) * len(shape))
    return pl.pallas_call(
        _fused_kernel,
        grid=(B, NT),
        in_specs=[
            pl.BlockSpec((1, TILE_N, D), lambda b, n: (b, n, 0)),
            const((D, D)), const((D, D)), const((D, D)), const((D, D)),
            const((D, D)), const((1, D)),
            const((D, D)), const((1, D)),
            const((D, OUT)), const((1, OUT)),
        ],
        out_specs=const((B, OUT)),
        out_shape=jax.ShapeDtypeStruct((B, OUT), jnp.float32),
        scratch_shapes=[pltpu.VMEM((8, D), jnp.float32)],
        compiler_params=pltpu.CompilerParams(
            dimension_semantics=("arbitrary", "arbitrary")),
    )(h_0, W_in, W_h1, W_h2, W_out,
      C1_w, C1_b.reshape(1, D), C2_w, C2_b.reshape(1, D),
      C3_w, C3_b.reshape(1, OUT))
